# Initial kernel scaffold; baseline (speedup 1.0000x reference)
import functools

import jax
import jax.numpy as jnp
from jax import lax
from jax.experimental import pallas as pl
from jax.experimental.pallas import tpu as pltpu
from jax.experimental.pallas import tpu_sc as plsc

V = 10000
F = 128
E = 320000
NC = 2
NS = 16
NW = NC * NS
CH = 128
NCH = 80
EPT = CH * NCH
EPAD = NW * EPT
RPT = V // NS


def _sc_scatter_body(x_hbm, src_hbm, dst_hbm, zero_hbm, out_hbm,
                     src_v, dst_v, rows_v, sem0, sem1, acc):
    c = lax.axis_index("c")
    s = lax.axis_index("s")
    wid = c * NS + s
    pltpu.sync_copy(zero_hbm.at[pl.ds(s * RPT, RPT)],
                    acc.at[pl.ds(s * RPT, RPT)])
    pltpu.sync_copy(src_hbm.at[wid], src_v)
    pltpu.sync_copy(dst_hbm.at[wid], dst_v)
    sems = (sem0, sem1)
    for b in range(2):
        pltpu.async_copy(x_hbm.at[src_v.at[b]], rows_v.at[b], sems[b])
    plsc.subcore_barrier()

    @pl.loop(0, NCH // 2 - 1)
    def _(i):
        for b in range(2):
            j = i * 2 + b
            pltpu.make_async_copy(x_hbm.at[src_v.at[j]], rows_v.at[b],
                                  sems[b]).wait()
            pltpu.sync_copy(rows_v.at[b], acc.at[dst_v.at[j]], add=True)
            pltpu.async_copy(x_hbm.at[src_v.at[j + 2]], rows_v.at[b], sems[b])

    for b in range(2):
        j = NCH - 2 + b
        pltpu.make_async_copy(x_hbm.at[src_v.at[j]], rows_v.at[b],
                              sems[b]).wait()
        pltpu.sync_copy(rows_v.at[b], acc.at[dst_v.at[j]], add=True)
    plsc.subcore_barrier()
    pltpu.sync_copy(acc.at[pl.ds(s * RPT, RPT)],
                    out_hbm.at[c, pl.ds(s * RPT, RPT)])


def _tc_combine_body(x_ref, p_ref, wst_ref, wnt_ref, b_ref, g_ref, be_ref,
                     o_ref):
    x = x_ref[...]
    agg = p_ref[0] + p_ref[1]
    h = (jnp.dot(x, wst_ref[...], preferred_element_type=jnp.float32)
         + jnp.dot(agg, wnt_ref[...], preferred_element_type=jnp.float32)
         + b_ref[...])
    mean = jnp.mean(h, axis=0, keepdims=True)
    d = h - mean
    var = jnp.mean(d * d, axis=0, keepdims=True)
    hn = d * lax.rsqrt(var + 1e-5)
    o_ref[...] = jnp.maximum(hn * g_ref[...] + be_ref[...], 0.0)


def kernel(X, edge_index, W_node, W_self, b_self, gamma, beta):
    x2 = X.reshape(V, F)
    xpad = jnp.concatenate([x2, jnp.zeros((8, F), x2.dtype)], axis=0)
    pad_n = EPAD - E
    src = jnp.concatenate(
        [edge_index[:, 0], jnp.full((pad_n,), V, jnp.int32)])
    dst = jnp.concatenate(
        [edge_index[:, 1], jnp.zeros((pad_n,), jnp.int32)])
    src3 = src.reshape(NW, NCH, CH)
    dst3 = dst.reshape(NW, NCH, CH)
    zeros_vf = jnp.zeros((V, F), jnp.float32)

    scatter = pl.kernel(
        _sc_scatter_body,
        out_type=jax.ShapeDtypeStruct((NC, V, F), jnp.float32),
        mesh=plsc.VectorSubcoreMesh(core_axis_name="c", subcore_axis_name="s"),
        scratch_types=[
            pltpu.VMEM((NCH, CH), jnp.int32),
            pltpu.VMEM((NCH, CH), jnp.int32),
            pltpu.VMEM((2, CH, F), jnp.float32),
            pltpu.SemaphoreType.DMA,
            pltpu.SemaphoreType.DMA,
            pltpu.VMEM_SHARED((V, F), jnp.float32),
        ],
    )
    partials = scatter(xpad, src3, dst3, zeros_vf)

    out2 = pl.pallas_call(
        _tc_combine_body,
        out_shape=jax.ShapeDtypeStruct((V, F), jnp.float32),
    )(x2, partials, W_self.T, W_node.T, b_self.reshape(1, F),
      gamma.reshape(1, F), beta.reshape(1, F))
    return out2.reshape(1, V, F)


# speedup vs baseline: 5.2924x; 5.2924x over previous
import jax
import jax.numpy as jnp
from jax import lax
from jax.experimental import pallas as pl
from jax.experimental.pallas import tpu as pltpu
from jax.experimental.pallas import tpu_sc as plsc

V = 10000
F = 128
FH = F // 2
E = 320000
NC = 2
NS = 16
CH = 128
NCH = 160
EPT = CH * NCH
EPAD = NS * EPT
VP = 10112
RPT = VP // NS


def _sc_scatter_body(x_hbm, src_hbm, dst_hbm, zero_hbm, out_hbm,
                     src_v, dst_v, rows_v, sem0, sem1, acc):
    c = lax.axis_index("c")
    s = lax.axis_index("s")
    pltpu.sync_copy(zero_hbm.at[pl.ds(s * RPT, RPT)],
                    acc.at[pl.ds(s * RPT, RPT)])
    pltpu.sync_copy(src_hbm.at[s], src_v)
    pltpu.sync_copy(dst_hbm.at[s], dst_v)
    sems = (sem0, sem1)
    xc = x_hbm.at[c]
    for b in range(2):
        pltpu.async_copy(xc.at[src_v.at[b]], rows_v.at[b], sems[b])
    plsc.subcore_barrier()

    @pl.loop(0, NCH // 2 - 1)
    def _(i):
        for b in range(2):
            j = i * 2 + b
            pltpu.make_async_copy(xc.at[src_v.at[j]], rows_v.at[b],
                                  sems[b]).wait()
            pltpu.sync_copy(rows_v.at[b], acc.at[dst_v.at[j]], add=True)
            pltpu.async_copy(xc.at[src_v.at[j + 2]], rows_v.at[b], sems[b])

    for b in range(2):
        j = NCH - 2 + b
        pltpu.make_async_copy(xc.at[src_v.at[j]], rows_v.at[b],
                              sems[b]).wait()
        pltpu.sync_copy(rows_v.at[b], acc.at[dst_v.at[j]], add=True)
    plsc.subcore_barrier()
    pltpu.sync_copy(acc.at[pl.ds(s * RPT, RPT)],
                    out_hbm.at[c, pl.ds(s * RPT, RPT)])


def _tc_combine_body(x_ref, p_ref, wst_ref, wnt_ref, b_ref, g_ref, be_ref,
                     o_ref):
    x = x_ref[...]
    agg = jnp.concatenate([p_ref[0], p_ref[1]], axis=-1)
    h = (jnp.dot(x, wst_ref[...], preferred_element_type=jnp.float32)
         + jnp.dot(agg, wnt_ref[...], preferred_element_type=jnp.float32)
         + b_ref[...])
    mean = jnp.mean(h, axis=0, keepdims=True)
    d = h - mean
    var = jnp.mean(d * d, axis=0, keepdims=True)
    hn = d * lax.rsqrt(var + 1e-5)
    o_ref[...] = jnp.maximum(hn * g_ref[...] + be_ref[...], 0.0)


def kernel(X, edge_index, W_node, W_self, b_self, gamma, beta):
    x2 = X.reshape(V, F)
    xpad = jnp.concatenate([x2, jnp.zeros((8, F), x2.dtype)], axis=0)
    xsplit = jnp.stack([xpad[:, :FH], xpad[:, FH:]])
    pad_n = EPAD - E
    src = jnp.concatenate(
        [edge_index[:, 0], jnp.full((pad_n,), V, jnp.int32)])
    dst = jnp.concatenate(
        [edge_index[:, 1], jnp.zeros((pad_n,), jnp.int32)])
    src3 = src.reshape(NS, NCH, CH)
    dst3 = dst.reshape(NS, NCH, CH)
    zeros_vf = jnp.zeros((VP, FH), jnp.float32)

    scatter = pl.kernel(
        _sc_scatter_body,
        out_type=jax.ShapeDtypeStruct((NC, VP, FH), jnp.float32),
        mesh=plsc.VectorSubcoreMesh(core_axis_name="c", subcore_axis_name="s"),
        scratch_types=[
            pltpu.VMEM((NCH, CH), jnp.int32),
            pltpu.VMEM((NCH, CH), jnp.int32),
            pltpu.VMEM((2, CH, FH), jnp.float32),
            pltpu.SemaphoreType.DMA,
            pltpu.SemaphoreType.DMA,
            pltpu.VMEM_SHARED((VP, FH), jnp.float32),
        ],
        compiler_params=pltpu.CompilerParams(use_tc_tiling_on_sc=False),
    )
    partials = scatter(xsplit, src3, dst3, zeros_vf)

    out2 = pl.pallas_call(
        _tc_combine_body,
        out_shape=jax.ShapeDtypeStruct((V, F), jnp.float32),
        grid=(1,),
        in_specs=[
            pl.BlockSpec((V, F), lambda i: (0, 0)),
            pl.BlockSpec((NC, V, FH), lambda i: (0, 0, 0)),
            pl.BlockSpec((F, F), lambda i: (0, 0)),
            pl.BlockSpec((F, F), lambda i: (0, 0)),
            pl.BlockSpec((1, F), lambda i: (0, 0)),
            pl.BlockSpec((1, F), lambda i: (0, 0)),
            pl.BlockSpec((1, F), lambda i: (0, 0)),
        ],
        out_specs=pl.BlockSpec((V, F), lambda i: (0, 0)),
    )(x2, partials, W_self.T, W_node.T, b_self.reshape(1, F),
      gamma.reshape(1, F), beta.reshape(1, F))
    return out2.reshape(1, V, F)
